# Initial kernel scaffold; baseline (speedup 1.0000x reference)
import jax
import jax.numpy as jnp
from jax.experimental import pallas as pl


def kernel(target, pred, latent, R_xyz):
    raise NotImplementedError("write your pallas kernel here")


# speedup vs baseline: 1.7965x; 1.7965x over previous
import functools

import jax
import jax.numpy as jnp
from jax import lax
from jax.experimental import pallas as pl
from jax.experimental.pallas import tpu as pltpu
from jax.experimental.pallas import tpu_sc as plsc

LAT_N = 65536
N_TOP = 20
NUM_TILES = 16
CHUNK = LAT_N // NUM_TILES
VREGS = CHUNK // 16
CAND = 32
NCAND = NUM_TILES * CAND
NEG = float("-inf")
IMAX = 2147483647

ROWS = 16384
COLS = 2048
BLOCK_ROWS = 512


def _sc_body(lat_hbm, rxyz_hbm, out_hbm,
             lat_v, xyz_v, loc_val_v, loc_idx_v, loc_xyz_v,
             cand_val_s, cand_idx_s, cand_xyz_s,
             mrg_val_v, mrg_idx_v, mrg_xyz_v, out_v):
    wid = lax.axis_index("s")
    base = wid * CHUNK
    lanes = lax.iota(jnp.int32, 16)

    pltpu.sync_copy(lat_hbm.at[pl.ds(base, CHUNK)], lat_v)
    pltpu.sync_copy(rxyz_hbm.at[:, pl.ds(base, CHUNK)], xyz_v)

    selv0 = jnp.full((16,), NEG, jnp.float32)
    selv1 = jnp.full((16,), NEG, jnp.float32)
    seli0 = jnp.full((16,), base, jnp.int32)
    seli1 = jnp.full((16,), base, jnp.int32)

    def scan_body(i, carry):
        bv, bi = carry
        v = lat_v[pl.ds(i * 16, 16)]
        gi = base + i * 16 + lanes
        upd = v > bv
        return jnp.where(upd, v, bv), jnp.where(upd, gi, bi)

    for k in range(N_TOP):
        bv, bi = lax.fori_loop(
            0, VREGS, scan_body,
            (jnp.full((16,), NEG, jnp.float32), jnp.full((16,), base, jnp.int32)))
        m = jnp.max(bv)
        eq = bv == m
        gsel = jnp.min(jnp.where(eq, bi, IMAX))
        if k < 16:
            selv0 = jnp.where(lanes == k, m, selv0)
            seli0 = jnp.where(lanes == k, gsel, seli0)
        else:
            selv1 = jnp.where(lanes == (k - 16), m, selv1)
            seli1 = jnp.where(lanes == (k - 16), gsel, seli1)
        plsc.store_scatter(lat_v, [jnp.full((16,), gsel - base, jnp.int32)],
                           jnp.full((16,), NEG, jnp.float32), mask=lanes == 0)

    loc_val_v[pl.ds(0, 16)] = selv0
    loc_val_v[pl.ds(16, 16)] = selv1
    loc_idx_v[pl.ds(0, 16)] = seli0
    loc_idx_v[pl.ds(16, 16)] = seli1

    off0 = seli0 - base
    off1 = seli1 - base
    for r in range(3):
        rvec = jnp.full((16,), r, jnp.int32)
        loc_xyz_v[pl.ds(r * CAND, 16)] = plsc.load_gather(xyz_v, [rvec, off0])
        loc_xyz_v[pl.ds(r * CAND + 16, 16)] = plsc.load_gather(xyz_v, [rvec, off1])

    pltpu.sync_copy(loc_val_v, cand_val_s.at[pl.ds(wid * CAND, CAND)])
    pltpu.sync_copy(loc_idx_v, cand_idx_s.at[pl.ds(wid * CAND, CAND)])
    for r in range(3):
        pltpu.sync_copy(loc_xyz_v.at[pl.ds(r * CAND, CAND)],
                        cand_xyz_s.at[pl.ds(r * NCAND + wid * CAND, CAND)])

    plsc.subcore_barrier()

    @pl.when(wid == 0)
    def _merge():
        pltpu.sync_copy(cand_val_s, mrg_val_v)
        pltpu.sync_copy(cand_idx_s, mrg_idx_v)
        pltpu.sync_copy(cand_xyz_s, mrg_xyz_v)

        def mscan(i, carry):
            bv, bgi, bp = carry
            v = mrg_val_v[pl.ds(i * 16, 16)]
            gi = mrg_idx_v[pl.ds(i * 16, 16)]
            p = i * 16 + lanes
            upd = (v > bv) | ((v == bv) & (gi < bgi))
            return (jnp.where(upd, v, bv), jnp.where(upd, gi, bgi),
                    jnp.where(upd, p, bp))

        selp0 = jnp.zeros((16,), jnp.int32)
        selp1 = jnp.zeros((16,), jnp.int32)
        for k in range(N_TOP):
            bv, bgi, bp = lax.fori_loop(
                0, NCAND // 16, mscan,
                (jnp.full((16,), NEG, jnp.float32),
                 jnp.full((16,), IMAX, jnp.int32),
                 jnp.zeros((16,), jnp.int32)))
            m = jnp.max(bv)
            eq = bv == m
            gsel = jnp.min(jnp.where(eq, bgi, IMAX))
            psel = jnp.min(jnp.where(eq & (bgi == gsel), bp, IMAX))
            if k < 16:
                selp0 = jnp.where(lanes == k, psel, selp0)
            else:
                selp1 = jnp.where(lanes == (k - 16), psel, selp1)
            plsc.store_scatter(mrg_val_v, [jnp.full((16,), psel, jnp.int32)],
                               jnp.full((16,), NEG, jnp.float32), mask=lanes == 0)

        valid1 = lanes < (N_TOP - 16)
        acc = jnp.zeros((16,), jnp.float32)
        for r in range(3):
            x0 = plsc.load_gather(mrg_xyz_v, [r * NCAND + selp0])
            x1 = plsc.load_gather(mrg_xyz_v, [r * NCAND + selp1])
            x1 = jnp.where(valid1, x1, jnp.float32(0.0))
            sv = x0 + x1
            qv = x0 * x0 + x1 * x1
            s1 = jnp.full((16,), jnp.sum(sv), jnp.float32)
            s2 = jnp.full((16,), jnp.sum(qv), jnp.float32)
            acc = acc + (s2 - s1 * s1 * jnp.float32(1.0 / N_TOP))
        out_v[...] = acc * jnp.float32(0.02 / (N_TOP - 1))
        pltpu.sync_copy(out_v, out_hbm)


_sc_topk_cov = functools.partial(
    pl.kernel,
    out_type=jax.ShapeDtypeStruct((16,), jnp.float32),
    mesh=plsc.VectorSubcoreMesh(core_axis_name="c", subcore_axis_name="s",
                                num_cores=1),
    compiler_params=pltpu.CompilerParams(needs_layout_passes=False),
    scratch_types=[
        pltpu.VMEM((CHUNK,), jnp.float32),
        pltpu.VMEM((3, CHUNK), jnp.float32),
        pltpu.VMEM((CAND,), jnp.float32),
        pltpu.VMEM((CAND,), jnp.int32),
        pltpu.VMEM((3 * CAND,), jnp.float32),
        pltpu.VMEM_SHARED((NCAND,), jnp.float32),
        pltpu.VMEM_SHARED((NCAND,), jnp.int32),
        pltpu.VMEM_SHARED((3 * NCAND,), jnp.float32),
        pltpu.VMEM((NCAND,), jnp.float32),
        pltpu.VMEM((NCAND,), jnp.int32),
        pltpu.VMEM((3 * NCAND,), jnp.float32),
        pltpu.VMEM((16,), jnp.float32),
    ],
)(_sc_body)


def _l1_body(t_ref, p_ref, o_ref):
    i = pl.program_id(0)

    @pl.when(i == 0)
    def _init():
        o_ref[...] = jnp.zeros((1, 1), jnp.float32)

    o_ref[...] += jnp.sum(jnp.abs(t_ref[...] - p_ref[...])).reshape(1, 1)

    @pl.when(i == pl.num_programs(0) - 1)
    def _norm():
        o_ref[...] = o_ref[...] * jnp.float32(1.0 / (ROWS * COLS))


_l1_mean = pl.pallas_call(
    _l1_body,
    grid=(ROWS // BLOCK_ROWS,),
    in_specs=[
        pl.BlockSpec((BLOCK_ROWS, COLS), lambda i: (i, 0)),
        pl.BlockSpec((BLOCK_ROWS, COLS), lambda i: (i, 0)),
    ],
    out_specs=pl.BlockSpec((1, 1), lambda i: (0, 0)),
    out_shape=jax.ShapeDtypeStruct((1, 1), jnp.float32),
)


def kernel(target, pred, latent, R_xyz):
    l1 = _l1_mean(target, pred)[0, 0]
    cov_term = _sc_topk_cov(latent, R_xyz)[0]
    return (l1 + cov_term, l1, cov_term)
